# Initial kernel scaffold; baseline (speedup 1.0000x reference)
import jax
import jax.numpy as jnp
from jax.experimental import pallas as pl


def kernel(x, tables):
    raise NotImplementedError("write your pallas kernel here")


# speedup vs baseline: 1.1286x; 1.1286x over previous
import functools

import jax
import jax.numpy as jnp
from jax import lax
from jax.experimental import pallas as pl
from jax.experimental.pallas import tpu as pltpu
from jax.experimental.pallas import tpu_sc as plsc

_NUM_FIELDS = 26
_VOCAB = 100000
_HIDDEN = 32
_L = 16
_NC = 2
_NS = 16
_CH = 128


@functools.cache
def _build(batch):
  nw = _NC * _NS
  bw = batch // nw
  nch = bw // _CH

  mesh = plsc.VectorSubcoreMesh(
      core_axis_name="c", subcore_axis_name="s",
      num_cores=_NC, num_subcores=_NS)

  @functools.partial(
      pl.kernel,
      out_type=jax.ShapeDtypeStruct((batch, _HIDDEN), jnp.float32),
      mesh=mesh,
      compiler_params=pltpu.CompilerParams(use_tc_tiling_on_sc=False),
      scratch_types=[
          pltpu.VMEM((_NUM_FIELDS, bw), jnp.int32),
          pltpu.VMEM((_CH,), jnp.int32),
          pltpu.VMEM((_CH,), jnp.int32),
          pltpu.VMEM((_CH, _HIDDEN), jnp.float32),
          pltpu.VMEM_SHARED((_NS * _CH, _HIDDEN), jnp.float32),
      ],
  )
  def enc(x_hbm, tab_hbm, out_hbm, xblk, gidx, sidx, rows, acc):
    c = lax.axis_index("c")
    s = lax.axis_index("s")
    wid = s * _NC + c
    base = wid * bw

    pltpu.sync_copy(x_hbm.at[:, pl.ds(base, bw)], xblk)

    for k in range(_CH // _L):
      sidx[pl.ds(k * _L, _L)] = lax.iota(jnp.int32, _L) + (s * _CH + k * _L)

    for ch in range(nch):
      row0 = ch * _CH

      def do_field(i, add):
        for k in range(_CH // _L):
          v = xblk[i, pl.ds(row0 + k * _L, _L)]
          gidx[pl.ds(k * _L, _L)] = v + i * _VOCAB
        pltpu.sync_copy(tab_hbm.at[gidx], rows)
        pltpu.sync_copy(rows, acc.at[sidx], add=add)

      do_field(0, False)

      def body(i, carry):
        do_field(i, True)
        return carry

      lax.fori_loop(1, _NUM_FIELDS, body, 0)

      pltpu.sync_copy(acc.at[pl.ds(s * _CH, _CH), :],
                      out_hbm.at[pl.ds(base + row0, _CH), :])

  return enc


@jax.jit
def kernel(x, tables):
  tab2d = tables.reshape(_NUM_FIELDS * _VOCAB, _HIDDEN)
  xt = x.astype(jnp.int32).T
  return _build(x.shape[0])(xt, tab2d)


# speedup vs baseline: 1.1727x; 1.0391x over previous
import functools

import jax
import jax.numpy as jnp
from jax import lax
from jax.experimental import pallas as pl
from jax.experimental.pallas import tpu as pltpu
from jax.experimental.pallas import tpu_sc as plsc

_NUM_FIELDS = 26
_VOCAB = 100000
_HIDDEN = 32
_L = 16
_NC = 2
_NS = 16
_CH = 128


@functools.cache
def _build(batch):
  nw = _NC * _NS
  bw = batch // nw
  nch = bw // _CH

  mesh = plsc.VectorSubcoreMesh(
      core_axis_name="c", subcore_axis_name="s",
      num_cores=_NC, num_subcores=_NS)

  @functools.partial(
      pl.kernel,
      out_type=jax.ShapeDtypeStruct((batch, _HIDDEN), jnp.float32),
      mesh=mesh,
      compiler_params=pltpu.CompilerParams(use_tc_tiling_on_sc=False),
      scratch_types=[
          pltpu.VMEM((_NUM_FIELDS, bw), jnp.int32),
          pltpu.VMEM((2, _CH), jnp.int32),
          pltpu.VMEM((_CH,), jnp.int32),
          pltpu.VMEM((2, _CH, _HIDDEN), jnp.float32),
          pltpu.VMEM_SHARED((_NS * _CH, _HIDDEN), jnp.float32),
          pltpu.SemaphoreType.DMA((2,)),
      ],
  )
  def enc(x_hbm, tab_hbm, out_hbm, xblk, gidx, sidx, rows, acc, sem):
    c = lax.axis_index("c")
    s = lax.axis_index("s")
    wid = s * _NC + c
    base = wid * bw

    pltpu.sync_copy(x_hbm.at[:, pl.ds(base, bw)], xblk)

    for k in range(_CH // _L):
      sidx[pl.ds(k * _L, _L)] = lax.iota(jnp.int32, _L) + (s * _CH + k * _L)

    for ch in range(nch):
      row0 = ch * _CH

      def build(i, p):
        for k in range(_CH // _L):
          v = xblk[i, pl.ds(row0 + k * _L, _L)]
          gidx[p, pl.ds(k * _L, _L)] = v + i * _VOCAB

      def gather_start(p):
        pltpu.async_copy(tab_hbm.at[gidx.at[p]], rows.at[p], sem.at[p])

      def gather_wait(p):
        pltpu.make_async_copy(tab_hbm.at[gidx.at[p]], rows.at[p],
                              sem.at[p]).wait()

      def scatter(p, add):
        pltpu.sync_copy(rows.at[p], acc.at[sidx], add=add)

      build(0, 0)
      gather_start(0)
      build(1, 1)
      gather_start(1)
      gather_wait(0)
      scatter(0, False)

      def body(i, carry):
        p = jnp.bitwise_and(i, 1)
        q = 1 - p
        build(i, p)
        gather_start(p)
        gather_wait(q)
        scatter(q, True)
        return carry

      lax.fori_loop(2, _NUM_FIELDS, body, 0)

      gather_wait(1)
      scatter(1, True)

      pltpu.sync_copy(acc.at[pl.ds(s * _CH, _CH), :],
                      out_hbm.at[pl.ds(base + row0, _CH), :])

  return enc


@jax.jit
def kernel(x, tables):
  tab2d = tables.reshape(_NUM_FIELDS * _VOCAB, _HIDDEN)
  xt = x.astype(jnp.int32).T
  return _build(x.shape[0])(xt, tab2d)


# speedup vs baseline: 4.7516x; 4.0520x over previous
import functools

import jax
import jax.numpy as jnp
from jax import lax
from jax.experimental import pallas as pl
from jax.experimental.pallas import tpu as pltpu
from jax.experimental.pallas import tpu_sc as plsc

_NUM_FIELDS = 26
_VOCAB = 100000
_HIDDEN = 32
_L = 16
_NC = 2
_NS = 16
_BC = 8192
_U = 8


@functools.cache
def _build(batch):
  nbc = batch // _BC

  mesh = plsc.VectorSubcoreMesh(
      core_axis_name="c", subcore_axis_name="s",
      num_cores=_NC, num_subcores=_NS)

  @functools.partial(
      pl.kernel,
      out_type=jax.ShapeDtypeStruct((_HIDDEN, batch), jnp.float32),
      mesh=mesh,
      compiler_params=pltpu.CompilerParams(
          use_tc_tiling_on_sc=True, needs_layout_passes=False),
      scratch_types=[
          pltpu.VMEM((_VOCAB,), jnp.float32),
          pltpu.VMEM((_BC,), jnp.int32),
          pltpu.VMEM((batch,), jnp.float32),
      ],
  )
  def enc(tt_hbm, xt_hbm, out_hbm, rowbuf, xcol, outcol):
    c = lax.axis_index("c")
    s = lax.axis_index("s")
    h = s * _NC + c

    def field(f, first):
      pltpu.sync_copy(tt_hbm.at[f, h, :], rowbuf)
      for cidx in range(nbc):
        pltpu.sync_copy(xt_hbm.at[f, pl.ds(cidx * _BC, _BC)], xcol)

        def body(k, carry):
          for j in range(_U):
            o = (k * _U + j) * _L
            v = xcol[pl.ds(o, _L)]
            g = plsc.load_gather(rowbuf, [v])
            if first:
              outcol[pl.ds(cidx * _BC + o, _L)] = g
            else:
              plsc.addupdate(outcol.at[pl.ds(cidx * _BC + o, _L)], g)
          return carry

        lax.fori_loop(0, _BC // (_L * _U), body, 0)

    field(0, True)

    def fbody(f, carry):
      field(f, False)
      return carry

    lax.fori_loop(1, _NUM_FIELDS, fbody, 0)

    pltpu.sync_copy(outcol, out_hbm.at[h, :])

  return enc


@jax.jit
def kernel(x, tables):
  tt = jnp.transpose(tables, (0, 2, 1))
  xt = x.astype(jnp.int32).T
  out_t = _build(x.shape[0])(tt, xt)
  return out_t.T
